# baseline (speedup 1.0000x reference)
import functools

import jax
import jax.numpy as jnp
from jax import lax
from jax.experimental import pallas as pl
from jax.experimental.pallas import tpu as pltpu
from jax.experimental.pallas import tpu_sc as plsc

N = 10000
NPAD = 10016
D = 128
H = 9
C = 8
HC = 72
RW = 96
ADW = 16
E = 320000
EPRIME = E + N
NCORES = 2
NSUB = 16
NW = NCORES * NSUB
CHUNK = 96
ET = -(-(-(-EPRIME // NW)) // CHUNK) * CHUNK
EP = ET * NW
NCH = ET // CHUNK
ZR = NPAD // NSUB
NB = 4
TBLK = NPAD // NB


def _tables_body(x_ref, wf_ref, wad_ref, t_ref, ad_ref):
    x = x_ref[...]
    t_ref[...] = jnp.dot(x, wf_ref[...], preferred_element_type=jnp.float32)
    ad_ref[...] = jnp.dot(x, wad_ref[...], preferred_element_type=jnp.float32)


def _build_tables(xpad, wfull, wad):
    return pl.pallas_call(
        _tables_body,
        grid=(NB,),
        in_specs=[
            pl.BlockSpec((TBLK, D), lambda i: (i, 0)),
            pl.BlockSpec((D, RW), lambda i: (0, 0)),
            pl.BlockSpec((D, ADW), lambda i: (0, 0)),
        ],
        out_specs=[
            pl.BlockSpec((TBLK, RW), lambda i: (i, 0)),
            pl.BlockSpec((TBLK, ADW), lambda i: (i, 0)),
        ],
        out_shape=[
            jax.ShapeDtypeStruct((NPAD, RW), jnp.float32),
            jax.ShapeDtypeStruct((NPAD, ADW), jnp.float32),
        ],
    )(xpad, wfull, wad)


_GD = lax.GatherDimensionNumbers(
    offset_dims=(), collapsed_slice_dims=(0,), start_index_map=(0,))


def _vgather(w, idx):
    return lax.gather(w, idx[:, None], _GD, slice_sizes=(1,),
                      mode=lax.GatherScatterMode.PROMISE_IN_BOUNDS)
def _edge_kernel(t1, ad1, s1, d1, t2, ad2, s2, d2, t3, ad3, s3, d3, zsrc,
                 p_out, sidx2, didx2, rows_t, rows_ad, val, acc,
                 semg0, semg1, sems0, sems1):
    cid = lax.axis_index("c")
    sid = lax.axis_index("s")
    wid = cid * NSUB + sid

    io = lax.iota(jnp.int32, 16)
    i0 = io // 8
    i1 = i0 + 2
    i2 = i0 + 4
    i3 = i0 + 6
    i4 = jnp.full((16,), 8, jnp.int32)
    lt8 = io < 8
    lt9 = io < 9
    semg = (semg0, semg1)
    sems = (sems0, sems1)

    for g, (tg, adg, sg, dg) in enumerate(
            ((t1, ad1, s1, d1), (t2, ad2, s2, d2), (t3, ad3, s3, d3))):
        pltpu.sync_copy(zsrc, acc.at[pl.ds(sid * ZR, ZR)])
        pltpu.sync_copy(sg.at[wid], sidx2)
        pltpu.sync_copy(dg.at[wid], didx2)
        plsc.subcore_barrier()

        def issue(ch, b, tg=tg, adg=adg):
            pltpu.async_copy(tg.at[sidx2.at[ch]], rows_t.at[b], semg[b])
            pltpu.async_copy(adg.at[didx2.at[ch]], rows_ad.at[b], semg[b])

        def wait_gather(b, tg=tg, adg=adg):
            pltpu.make_async_copy(tg.at[pl.ds(0, CHUNK)],
                                  rows_t.at[b], semg[b]).wait()
            pltpu.make_async_copy(adg.at[pl.ds(0, CHUNK)],
                                  rows_ad.at[b], semg[b]).wait()

        def drain_scatter(b, tg=tg):
            pltpu.make_async_copy(tg.at[pl.ds(0, CHUNK)],
                                  val.at[b], sems[b]).wait()

        def compute(b):
            rt = rows_t.at[b]
            ra = rows_ad.at[b]
            vb = val.at[b]

            @plsc.parallel_loop(0, CHUNK, unroll=8)
            def edge_body(e):
                a_s = rt[e, pl.ds(72, 16)]
                a_d = ra[e, :]
                att = a_s + a_d
                att = jnp.maximum(att, 0.2 * att)
                w = jnp.exp(att)
                w0 = _vgather(w, i0)
                w1 = _vgather(w, i1)
                w2 = _vgather(w, i2)
                w3 = _vgather(w, i3)
                w4 = _vgather(w, i4)
                vb[e, pl.ds(0, 16)] = rt[e, pl.ds(0, 16)] * w0
                vb[e, pl.ds(16, 16)] = rt[e, pl.ds(16, 16)] * w1
                vb[e, pl.ds(32, 16)] = rt[e, pl.ds(32, 16)] * w2
                vb[e, pl.ds(48, 16)] = rt[e, pl.ds(48, 16)] * w3
                vb[e, pl.ds(64, 16)] = jnp.where(
                    lt8, rt[e, pl.ds(64, 16)] * w4, 0.0)
                vb[e, pl.ds(80, 16)] = jnp.where(lt9, w, 0.0)

        def scatter(ch, b):
            pltpu.async_copy(val.at[b], acc.at[didx2.at[ch]], sems[b],
                             add=True)

        issue(0, 0)

        def pair_body(i, carry):
            ch0 = 2 * i
            issue(ch0 + 1, 1)
            wait_gather(0)

            @pl.when(i >= 1)
            def _():
                drain_scatter(0)
            compute(0)
            scatter(ch0, 0)

            @pl.when(ch0 + 2 < NCH)
            def _():
                issue(ch0 + 2, 0)
            wait_gather(1)

            @pl.when(i >= 1)
            def _():
                drain_scatter(1)
            compute(1)
            scatter(ch0 + 1, 1)
            return carry

        lax.fori_loop(0, NCH // 2, pair_body, 0)
        if NCH % 2:
            wait_gather(0)
            drain_scatter(0)
            compute(0)
            scatter(NCH - 1, 0)
        drain_scatter(0)
        drain_scatter(1)
        plsc.subcore_barrier()
        pltpu.sync_copy(acc.at[pl.ds(sid * ZR, ZR)],
                        p_out.at[g, cid, pl.ds(sid * ZR, ZR)])


def _run_edges(tabs, zsrc):
    mesh = plsc.VectorSubcoreMesh(core_axis_name="c", subcore_axis_name="s",
                                  num_cores=NCORES, num_subcores=NSUB)
    k = pl.kernel(
        _edge_kernel,
        mesh=mesh,
        compiler_params=pltpu.CompilerParams(use_tc_tiling_on_sc=False,
                                             needs_layout_passes=False),
        out_type=jax.ShapeDtypeStruct((3, NCORES, NPAD, RW), jnp.float32),
        scratch_types=[
            pltpu.VMEM((NCH, CHUNK), jnp.int32),
            pltpu.VMEM((NCH, CHUNK), jnp.int32),
            pltpu.VMEM((2, CHUNK, RW), jnp.float32),
            pltpu.VMEM((2, CHUNK, ADW), jnp.float32),
            pltpu.VMEM((2, CHUNK, RW), jnp.float32),
            pltpu.VMEM_SHARED((NPAD, RW), jnp.float32),
            pltpu.SemaphoreType.DMA,
            pltpu.SemaphoreType.DMA,
            pltpu.SemaphoreType.DMA,
            pltpu.SemaphoreType.DMA,
        ],
    )
    return k(*tabs, zsrc)


def _final_body(p_ref, bias_ref, exp9_ref, fw_ref, fb_ref, o_ref):
    p = p_ref[...]
    outs = []
    for g in range(3):
        num = p[g, 0, :, 0:HC] + p[g, 1, :, 0:HC]
        den = p[g, 0, :, 80:89] + p[g, 1, :, 80:89]
        rec = 1.0 / (den + 1e-16)
        rec_exp = jnp.dot(rec, exp9_ref[...],
                          preferred_element_type=jnp.float32)
        outs.append(num * rec_exp + bias_ref[g])
    xcat = jnp.concatenate(outs, axis=1)
    xcat = jnp.maximum(xcat, 0.0)
    logits = jnp.dot(xcat, fw_ref[...],
                     preferred_element_type=jnp.float32) + fb_ref[...]
    m = jnp.max(logits, axis=1, keepdims=True)
    ex = jnp.exp(logits - m)
    o_ref[...] = ex / jnp.sum(ex, axis=1, keepdims=True)


def _finalize(p, biases, exp9, fnn_w, fnn_b):
    blk = 2000
    return pl.pallas_call(
        _final_body,
        grid=(5,),
        in_specs=[
            pl.BlockSpec((3, NCORES, blk, RW), lambda i: (0, 0, i, 0)),
            pl.BlockSpec((3, HC), lambda i: (0, 0)),
            pl.BlockSpec((H, HC), lambda i: (0, 0)),
            pl.BlockSpec((3 * HC, 2), lambda i: (0, 0)),
            pl.BlockSpec((1, 2), lambda i: (0, 0)),
        ],
        out_specs=pl.BlockSpec((blk, 2), lambda i: (i, 0)),
        out_shape=jax.ShapeDtypeStruct((N, 2), jnp.float32),
    )(p, biases, exp9, fnn_w, fnn_b)


def _att_fold(att):
    a = att.reshape(H, C)
    return (a[:, :, None] * jnp.eye(H, dtype=jnp.float32)[:, None, :]
            ).reshape(HC, H)


def _edge_arrays(ei):
    loop = jnp.arange(N, dtype=jnp.int32)
    src = jnp.concatenate([ei[0].astype(jnp.int32), loop,
                           jnp.zeros((EP - EPRIME,), jnp.int32)])
    dst = jnp.concatenate([ei[1].astype(jnp.int32), loop,
                           jnp.full((EP - EPRIME,), N, jnp.int32)])
    return src.reshape(NW, NCH, CHUNK), dst.reshape(NW, NCH, CHUNK)


def kernel(x1, edge_index1, x2, edge_index2, x3, edge_index3,
           W1, att_src1, att_dst1, b1,
           W2, att_src2, att_dst2, b2,
           W3, att_src3, att_dst3, b3,
           fnn_W, fnn_b):
    pad = jnp.zeros((NPAD - N, D), jnp.float32)
    tabs = []
    for x, ei, W, a_s, a_d in ((x1, edge_index1, W1, att_src1, att_dst1),
                               (x2, edge_index2, W2, att_src2, att_dst2),
                               (x3, edge_index3, W3, att_src3, att_dst3)):
        ms = jnp.pad(_att_fold(a_s), ((0, 0), (0, RW - HC - H)))
        md = jnp.pad(_att_fold(a_d), ((0, 0), (0, ADW - H)))
        wfull = jnp.concatenate([W, W @ ms], axis=1)
        wad = W @ md
        xpad = jnp.concatenate([x, pad], axis=0)
        t, ad = _build_tables(xpad, wfull, wad)
        s, d = _edge_arrays(ei)
        tabs.extend([t, ad, s, d])

    zsrc = jnp.zeros((ZR, RW), jnp.float32)
    p = _run_edges(tabs, zsrc)

    biases = jnp.stack([b1, b2, b3])
    exp9 = jnp.repeat(jnp.eye(H, dtype=jnp.float32), C, axis=1)
    return _finalize(p, biases, exp9, fnn_W, fnn_b.reshape(1, 2))


# speedup vs baseline: 1.1520x; 1.1520x over previous
import functools

import numpy as np

import jax
import jax.numpy as jnp
from jax import lax
from jax.experimental import pallas as pl
from jax.experimental.pallas import tpu as pltpu
from jax.experimental.pallas import tpu_sc as plsc

N = 10000
NPAD = 10016
D = 128
H = 9
C = 8
HC = 72
RW = 96
ADW = 16
E = 320000
EPRIME = E + N
NCORES = 2
NSUB = 16
NW = NCORES * NSUB
CHUNK = 96
ET = -(-(-(-EPRIME // NW)) // CHUNK) * CHUNK
EP = ET * NW
NCH = ET // CHUNK
ZR = NPAD // NSUB
TW = 48
NB = 4
TBLK = NPAD // NB


def _tables_body(x_ref, wf_ref, wad_ref, t_ref, ad_ref):
    x = x_ref[...]
    y = jnp.dot(x, wf_ref[...], preferred_element_type=jnp.float32)
    t_ref[...] = y.astype(jnp.bfloat16)
    ad_ref[...] = jnp.dot(x, wad_ref[...], preferred_element_type=jnp.float32)


def _build_tables(xpad, wfull, wad):
    return pl.pallas_call(
        _tables_body,
        grid=(NB,),
        in_specs=[
            pl.BlockSpec((TBLK, D), lambda i: (i, 0)),
            pl.BlockSpec((D, RW), lambda i: (0, 0)),
            pl.BlockSpec((D, ADW), lambda i: (0, 0)),
        ],
        out_specs=[
            pl.BlockSpec((TBLK, RW), lambda i: (i, 0)),
            pl.BlockSpec((TBLK, ADW), lambda i: (i, 0)),
        ],
        out_shape=[
            jax.ShapeDtypeStruct((NPAD, RW), jnp.bfloat16),
            jax.ShapeDtypeStruct((NPAD, ADW), jnp.float32),
        ],
    )(xpad, wfull, wad)


_GD = lax.GatherDimensionNumbers(
    offset_dims=(), collapsed_slice_dims=(0,), start_index_map=(0,))


def _vgather(w, idx):
    return lax.gather(w, idx[:, None], _GD, slice_sizes=(1,),
                      mode=lax.GatherScatterMode.PROMISE_IN_BOUNDS)
def _edge_kernel(t1, ad1, s1, d1, t2, ad2, s2, d2, t3, ad3, s3, d3, zsrc,
                 p_out, sidx2, didx2, rows_t, rows_ad, val, acc,
                 semg0, semg1, sems0, sems1):
    cid = lax.axis_index("c")
    sid = lax.axis_index("s")
    wid = cid * NSUB + sid

    io = lax.iota(jnp.int32, 16)
    i0 = io // 8
    i1 = i0 + 2
    i2 = i0 + 4
    i3 = i0 + 6
    i4 = jnp.full((16,), 8, jnp.int32)
    lt8 = io < 8
    lt9 = io < 9
    semg = (semg0, semg1)
    sems = (sems0, sems1)

    for g, (tg, adg, sg, dg) in enumerate(
            ((t1, ad1, s1, d1), (t2, ad2, s2, d2), (t3, ad3, s3, d3))):
        pltpu.sync_copy(zsrc, acc.at[pl.ds(sid * ZR, ZR)])
        pltpu.sync_copy(sg.at[wid], sidx2)
        pltpu.sync_copy(dg.at[wid], didx2)
        plsc.subcore_barrier()

        def issue(ch, b, tg=tg, adg=adg):
            pltpu.async_copy(tg.at[sidx2.at[ch]], rows_t.at[b], semg[b])
            pltpu.async_copy(adg.at[didx2.at[ch]], rows_ad.at[b], semg[b])

        def wait_gather(b, tg=tg, adg=adg):
            pltpu.make_async_copy(tg.at[pl.ds(0, CHUNK)],
                                  rows_t.at[b], semg[b]).wait()
            pltpu.make_async_copy(adg.at[pl.ds(0, CHUNK)],
                                  rows_ad.at[b], semg[b]).wait()

        def drain_scatter(b, tg=tg):
            pltpu.make_async_copy(tg.at[pl.ds(0, CHUNK)],
                                  val.at[b], sems[b]).wait()

        def compute(b):
            rt = rows_t.at[b]
            ra = rows_ad.at[b]
            vb = val.at[b]

            msk = jnp.int32(-65536)

            @plsc.parallel_loop(0, CHUNK, unroll=4)
            def edge_body(e):
                v01 = rt[e, pl.ds(0, 16)]
                v23 = rt[e, pl.ds(16, 16)]
                v45 = rt[e, pl.ds(32, 16)]
                g0 = plsc.bitcast(jnp.left_shift(v01, 16), jnp.float32)
                g1 = plsc.bitcast(jnp.bitwise_and(v01, msk), jnp.float32)
                g2 = plsc.bitcast(jnp.left_shift(v23, 16), jnp.float32)
                g3 = plsc.bitcast(jnp.bitwise_and(v23, msk), jnp.float32)
                g4 = plsc.bitcast(jnp.left_shift(v45, 16), jnp.float32)
                a_s = plsc.bitcast(jnp.bitwise_and(v45, msk), jnp.float32)
                a_d = ra[e, :]
                att = a_s + a_d
                att = jnp.maximum(att, 0.2 * att)
                w = jnp.exp(att)
                w0 = _vgather(w, i0)
                w1 = _vgather(w, i1)
                w2 = _vgather(w, i2)
                w3 = _vgather(w, i3)
                w4 = _vgather(w, i4)
                vb[e, pl.ds(0, 16)] = g0 * w0
                vb[e, pl.ds(16, 16)] = g1 * w1
                vb[e, pl.ds(32, 16)] = g2 * w2
                vb[e, pl.ds(48, 16)] = g3 * w3
                vb[e, pl.ds(64, 16)] = g4 * w4
                vb[e, pl.ds(80, 16)] = jnp.where(lt9, w, 0.0)

        def scatter(ch, b):
            pltpu.async_copy(val.at[b], acc.at[didx2.at[ch]], sems[b],
                             add=True)

        issue(0, 0)

        def pair_body(i, carry):
            ch0 = 2 * i
            issue(ch0 + 1, 1)
            wait_gather(0)

            @pl.when(i >= 1)
            def _():
                drain_scatter(0)
            compute(0)
            scatter(ch0, 0)

            @pl.when(ch0 + 2 < NCH)
            def _():
                issue(ch0 + 2, 0)
            wait_gather(1)

            @pl.when(i >= 1)
            def _():
                drain_scatter(1)
            compute(1)
            scatter(ch0 + 1, 1)
            return carry

        lax.fori_loop(0, NCH // 2, pair_body, 0)
        if NCH % 2:
            wait_gather(0)
            drain_scatter(0)
            compute(0)
            scatter(NCH - 1, 0)
        drain_scatter(0)
        drain_scatter(1)
        plsc.subcore_barrier()
        pltpu.sync_copy(acc.at[pl.ds(sid * ZR, ZR)],
                        p_out.at[g, cid, pl.ds(sid * ZR, ZR)])


def _run_edges(tabs, zsrc):
    mesh = plsc.VectorSubcoreMesh(core_axis_name="c", subcore_axis_name="s",
                                  num_cores=NCORES, num_subcores=NSUB)
    k = pl.kernel(
        _edge_kernel,
        mesh=mesh,
        compiler_params=pltpu.CompilerParams(use_tc_tiling_on_sc=False,
                                             needs_layout_passes=False),
        out_type=jax.ShapeDtypeStruct((3, NCORES, NPAD, RW), jnp.float32),
        scratch_types=[
            pltpu.VMEM((NCH, CHUNK), jnp.int32),
            pltpu.VMEM((NCH, CHUNK), jnp.int32),
            pltpu.VMEM((2, CHUNK, TW), jnp.int32),
            pltpu.VMEM((2, CHUNK, ADW), jnp.float32),
            pltpu.VMEM((2, CHUNK, RW), jnp.float32),
            pltpu.VMEM_SHARED((NPAD, RW), jnp.float32),
            pltpu.SemaphoreType.DMA,
            pltpu.SemaphoreType.DMA,
            pltpu.SemaphoreType.DMA,
            pltpu.SemaphoreType.DMA,
        ],
    )
    return k(*tabs, zsrc)


def _final_body(p_ref, bias_ref, exp9_ref, fw_ref, fb_ref, o_ref):
    p = p_ref[...]
    outs = []
    for g in range(3):
        num = p[g, 0, :, 0:HC] + p[g, 1, :, 0:HC]
        den = p[g, 0, :, 80:89] + p[g, 1, :, 80:89]
        rec = 1.0 / (den + 1e-16)
        rec_exp = jnp.dot(rec, exp9_ref[...],
                          preferred_element_type=jnp.float32)
        outs.append(num * rec_exp + bias_ref[g])
    xcat = jnp.concatenate(outs, axis=1)
    xcat = jnp.maximum(xcat, 0.0)
    logits = jnp.dot(xcat, fw_ref[...],
                     preferred_element_type=jnp.float32) + fb_ref[...]
    m = jnp.max(logits, axis=1, keepdims=True)
    ex = jnp.exp(logits - m)
    o_ref[...] = ex / jnp.sum(ex, axis=1, keepdims=True)


def _finalize(p, biases, exp9, fnn_w, fnn_b):
    blk = 2000
    return pl.pallas_call(
        _final_body,
        grid=(5,),
        in_specs=[
            pl.BlockSpec((3, NCORES, blk, RW), lambda i: (0, 0, i, 0)),
            pl.BlockSpec((3, HC), lambda i: (0, 0)),
            pl.BlockSpec((H, HC), lambda i: (0, 0)),
            pl.BlockSpec((3 * HC, 2), lambda i: (0, 0)),
            pl.BlockSpec((1, 2), lambda i: (0, 0)),
        ],
        out_specs=pl.BlockSpec((blk, 2), lambda i: (i, 0)),
        out_shape=jax.ShapeDtypeStruct((N, 2), jnp.float32),
    )(p, biases, exp9, fnn_w, fnn_b)


def _perm_matrix():
    p = np.zeros((RW, RW), np.float32)
    for k in range(16):
        p[k, 2 * k] = 1.0
        p[16 + k, 2 * k + 1] = 1.0
        p[32 + k, 32 + 2 * k] = 1.0
        p[48 + k, 32 + 2 * k + 1] = 1.0
    for k in range(8):
        p[64 + k, 64 + 2 * k] = 1.0
    for k in range(H):
        p[72 + k, 64 + 2 * k + 1] = 1.0
    return p


_PERM = _perm_matrix()


def _att_fold(att):
    a = att.reshape(H, C)
    return (a[:, :, None] * jnp.eye(H, dtype=jnp.float32)[:, None, :]
            ).reshape(HC, H)


def _edge_arrays(ei):
    loop = jnp.arange(N, dtype=jnp.int32)
    src = jnp.concatenate([ei[0].astype(jnp.int32), loop,
                           jnp.zeros((EP - EPRIME,), jnp.int32)])
    dst = jnp.concatenate([ei[1].astype(jnp.int32), loop,
                           jnp.full((EP - EPRIME,), N, jnp.int32)])
    return src.reshape(NW, NCH, CHUNK), dst.reshape(NW, NCH, CHUNK)


def kernel(x1, edge_index1, x2, edge_index2, x3, edge_index3,
           W1, att_src1, att_dst1, b1,
           W2, att_src2, att_dst2, b2,
           W3, att_src3, att_dst3, b3,
           fnn_W, fnn_b):
    pad = jnp.zeros((NPAD - N, D), jnp.float32)
    tabs = []
    for x, ei, W, a_s, a_d in ((x1, edge_index1, W1, att_src1, att_dst1),
                               (x2, edge_index2, W2, att_src2, att_dst2),
                               (x3, edge_index3, W3, att_src3, att_dst3)):
        ms = jnp.pad(_att_fold(a_s), ((0, 0), (0, RW - HC - H)))
        md = jnp.pad(_att_fold(a_d), ((0, 0), (0, ADW - H)))
        wfull = jnp.concatenate([W, W @ ms], axis=1) @ _PERM
        wad = W @ md
        xpad = jnp.concatenate([x, pad], axis=0)
        t, ad = _build_tables(xpad, wfull, wad)
        t32 = lax.bitcast_convert_type(t.reshape(NPAD, TW, 2), jnp.int32)
        s, d = _edge_arrays(ei)
        tabs.extend([t32, ad, s, d])

    zsrc = jnp.zeros((ZR, RW), jnp.float32)
    p = _run_edges(tabs, zsrc)

    biases = jnp.stack([b1, b2, b3])
    exp9 = jnp.repeat(jnp.eye(H, dtype=jnp.float32), C, axis=1)
    return _finalize(p, biases, exp9, fnn_W, fnn_b.reshape(1, 2))


# speedup vs baseline: 1.1744x; 1.0195x over previous
import functools

import numpy as np

import jax
import jax.numpy as jnp
from jax import lax
from jax.experimental import pallas as pl
from jax.experimental.pallas import tpu as pltpu
from jax.experimental.pallas import tpu_sc as plsc

N = 10000
NPAD = 10016
D = 128
H = 9
C = 8
HC = 72
RW = 96
ADW = 16
E = 320000
EPRIME = E + N
NCORES = 2
NSUB = 16
NW = NCORES * NSUB
CHUNK = 128
ET = -(-(-(-EPRIME // NW)) // CHUNK) * CHUNK
EP = ET * NW
NCH = ET // CHUNK
ZR = NPAD // NSUB
TW = 48
NB = 4
TBLK = NPAD // NB


def _tables_body(x_ref, wf_ref, wad_ref, t_ref, ad_ref):
    x = x_ref[...]
    y = jnp.dot(x, wf_ref[...], preferred_element_type=jnp.float32)
    t_ref[...] = y.astype(jnp.bfloat16)
    ad_ref[...] = jnp.dot(x, wad_ref[...], preferred_element_type=jnp.float32)


def _build_tables(xpad, wfull, wad):
    return pl.pallas_call(
        _tables_body,
        grid=(NB,),
        in_specs=[
            pl.BlockSpec((TBLK, D), lambda i: (i, 0)),
            pl.BlockSpec((D, RW), lambda i: (0, 0)),
            pl.BlockSpec((D, ADW), lambda i: (0, 0)),
        ],
        out_specs=[
            pl.BlockSpec((TBLK, RW), lambda i: (i, 0)),
            pl.BlockSpec((TBLK, ADW), lambda i: (i, 0)),
        ],
        out_shape=[
            jax.ShapeDtypeStruct((NPAD, RW), jnp.bfloat16),
            jax.ShapeDtypeStruct((NPAD, ADW), jnp.float32),
        ],
    )(xpad, wfull, wad)


_GD = lax.GatherDimensionNumbers(
    offset_dims=(), collapsed_slice_dims=(0,), start_index_map=(0,))


def _vgather(w, idx):
    return lax.gather(w, idx[:, None], _GD, slice_sizes=(1,),
                      mode=lax.GatherScatterMode.PROMISE_IN_BOUNDS)
def _edge_kernel(t1, ad1, s1, d1, t2, ad2, s2, d2, t3, ad3, s3, d3, zsrc,
                 p_out, sidx2, didx2, rows_t, rows_ad, val, acc,
                 semg0, semg1, sems0, sems1):
    cid = lax.axis_index("c")
    sid = lax.axis_index("s")
    wid = cid * NSUB + sid

    io = lax.iota(jnp.int32, 16)
    i0 = io // 8
    i1 = i0 + 2
    i2 = i0 + 4
    i3 = i0 + 6
    i4 = jnp.full((16,), 8, jnp.int32)
    lt8 = io < 8
    lt9 = io < 9
    semg = (semg0, semg1)
    sems = (sems0, sems1)

    for g, (tg, adg, sg, dg) in enumerate(
            ((t1, ad1, s1, d1), (t2, ad2, s2, d2), (t3, ad3, s3, d3))):
        pltpu.sync_copy(zsrc, acc.at[pl.ds(sid * ZR, ZR)])
        pltpu.sync_copy(sg.at[wid], sidx2)
        pltpu.sync_copy(dg.at[wid], didx2)
        plsc.subcore_barrier()

        def issue(ch, b, tg=tg, adg=adg):
            pltpu.async_copy(tg.at[sidx2.at[ch]], rows_t.at[b], semg[b])
            pltpu.async_copy(adg.at[didx2.at[ch]], rows_ad.at[b], semg[b])

        def wait_gather(b, tg=tg, adg=adg):
            pltpu.make_async_copy(tg.at[pl.ds(0, CHUNK)],
                                  rows_t.at[b], semg[b]).wait()
            pltpu.make_async_copy(adg.at[pl.ds(0, CHUNK)],
                                  rows_ad.at[b], semg[b]).wait()

        def drain_scatter(b, tg=tg):
            pltpu.make_async_copy(tg.at[pl.ds(0, CHUNK)],
                                  val.at[b], sems[b]).wait()

        def compute(b):
            rt = rows_t.at[b]
            ra = rows_ad.at[b]
            vb = val.at[b]

            msk = jnp.int32(-65536)

            @plsc.parallel_loop(0, CHUNK, unroll=4)
            def edge_body(e):
                v01 = rt[e, pl.ds(0, 16)]
                v23 = rt[e, pl.ds(16, 16)]
                v45 = rt[e, pl.ds(32, 16)]
                g0 = plsc.bitcast(jnp.left_shift(v01, 16), jnp.float32)
                g1 = plsc.bitcast(jnp.bitwise_and(v01, msk), jnp.float32)
                g2 = plsc.bitcast(jnp.left_shift(v23, 16), jnp.float32)
                g3 = plsc.bitcast(jnp.bitwise_and(v23, msk), jnp.float32)
                g4 = plsc.bitcast(jnp.left_shift(v45, 16), jnp.float32)
                a_s = plsc.bitcast(jnp.bitwise_and(v45, msk), jnp.float32)
                a_d = ra[e, :]
                att = a_s + a_d
                att = jnp.maximum(att, 0.2 * att)
                w = jnp.exp(att)
                w0 = _vgather(w, i0)
                w1 = _vgather(w, i1)
                w2 = _vgather(w, i2)
                w3 = _vgather(w, i3)
                w4 = _vgather(w, i4)
                vb[e, pl.ds(0, 16)] = g0 * w0
                vb[e, pl.ds(16, 16)] = g1 * w1
                vb[e, pl.ds(32, 16)] = g2 * w2
                vb[e, pl.ds(48, 16)] = g3 * w3
                vb[e, pl.ds(64, 16)] = g4 * w4
                vb[e, pl.ds(80, 16)] = jnp.where(lt9, w, 0.0)

        def scatter(ch, b):
            pltpu.async_copy(val.at[b], acc.at[didx2.at[ch]], sems[b],
                             add=True)

        issue(0, 0)

        def pair_body(i, carry):
            ch0 = 2 * i
            issue(ch0 + 1, 1)
            wait_gather(0)

            @pl.when(i >= 1)
            def _():
                drain_scatter(0)
            compute(0)
            scatter(ch0, 0)

            @pl.when(ch0 + 2 < NCH)
            def _():
                issue(ch0 + 2, 0)
            wait_gather(1)

            @pl.when(i >= 1)
            def _():
                drain_scatter(1)
            compute(1)
            scatter(ch0 + 1, 1)
            return carry

        lax.fori_loop(0, NCH // 2, pair_body, 0)
        if NCH % 2:
            wait_gather(0)
            drain_scatter(0)
            compute(0)
            scatter(NCH - 1, 0)
        drain_scatter(0)
        drain_scatter(1)
        plsc.subcore_barrier()
        pltpu.sync_copy(acc.at[pl.ds(sid * ZR, ZR)],
                        p_out.at[g, cid, pl.ds(sid * ZR, ZR)])


def _run_edges(tabs, zsrc):
    mesh = plsc.VectorSubcoreMesh(core_axis_name="c", subcore_axis_name="s",
                                  num_cores=NCORES, num_subcores=NSUB)
    k = pl.kernel(
        _edge_kernel,
        mesh=mesh,
        compiler_params=pltpu.CompilerParams(use_tc_tiling_on_sc=False,
                                             needs_layout_passes=False),
        out_type=jax.ShapeDtypeStruct((3, NCORES, NPAD, RW), jnp.float32),
        scratch_types=[
            pltpu.VMEM((NCH, CHUNK), jnp.int32),
            pltpu.VMEM((NCH, CHUNK), jnp.int32),
            pltpu.VMEM((2, CHUNK, TW), jnp.int32),
            pltpu.VMEM((2, CHUNK, ADW), jnp.float32),
            pltpu.VMEM((2, CHUNK, RW), jnp.float32),
            pltpu.VMEM_SHARED((NPAD, RW), jnp.float32),
            pltpu.SemaphoreType.DMA,
            pltpu.SemaphoreType.DMA,
            pltpu.SemaphoreType.DMA,
            pltpu.SemaphoreType.DMA,
        ],
    )
    return k(*tabs, zsrc)


def _final_body(p_ref, bias_ref, exp9_ref, fw_ref, fb_ref, o_ref):
    p = p_ref[...]
    outs = []
    for g in range(3):
        num = p[g, 0, :, 0:HC] + p[g, 1, :, 0:HC]
        den = p[g, 0, :, 80:89] + p[g, 1, :, 80:89]
        rec = 1.0 / (den + 1e-16)
        rec_exp = jnp.dot(rec, exp9_ref[...],
                          preferred_element_type=jnp.float32)
        outs.append(num * rec_exp + bias_ref[g])
    xcat = jnp.concatenate(outs, axis=1)
    xcat = jnp.maximum(xcat, 0.0)
    logits = jnp.dot(xcat, fw_ref[...],
                     preferred_element_type=jnp.float32) + fb_ref[...]
    m = jnp.max(logits, axis=1, keepdims=True)
    ex = jnp.exp(logits - m)
    o_ref[...] = ex / jnp.sum(ex, axis=1, keepdims=True)


def _finalize(p, biases, exp9, fnn_w, fnn_b):
    blk = 2000
    return pl.pallas_call(
        _final_body,
        grid=(5,),
        in_specs=[
            pl.BlockSpec((3, NCORES, blk, RW), lambda i: (0, 0, i, 0)),
            pl.BlockSpec((3, HC), lambda i: (0, 0)),
            pl.BlockSpec((H, HC), lambda i: (0, 0)),
            pl.BlockSpec((3 * HC, 2), lambda i: (0, 0)),
            pl.BlockSpec((1, 2), lambda i: (0, 0)),
        ],
        out_specs=pl.BlockSpec((blk, 2), lambda i: (i, 0)),
        out_shape=jax.ShapeDtypeStruct((N, 2), jnp.float32),
    )(p, biases, exp9, fnn_w, fnn_b)


def _perm_matrix():
    p = np.zeros((RW, RW), np.float32)
    for k in range(16):
        p[k, 2 * k] = 1.0
        p[16 + k, 2 * k + 1] = 1.0
        p[32 + k, 32 + 2 * k] = 1.0
        p[48 + k, 32 + 2 * k + 1] = 1.0
    for k in range(8):
        p[64 + k, 64 + 2 * k] = 1.0
    for k in range(H):
        p[72 + k, 64 + 2 * k + 1] = 1.0
    return p


_PERM = _perm_matrix()


def _att_fold(att):
    a = att.reshape(H, C)
    return (a[:, :, None] * jnp.eye(H, dtype=jnp.float32)[:, None, :]
            ).reshape(HC, H)


def _edge_arrays(ei):
    loop = jnp.arange(N, dtype=jnp.int32)
    src = jnp.concatenate([ei[0].astype(jnp.int32), loop,
                           jnp.zeros((EP - EPRIME,), jnp.int32)])
    dst = jnp.concatenate([ei[1].astype(jnp.int32), loop,
                           jnp.full((EP - EPRIME,), N, jnp.int32)])
    return src.reshape(NW, NCH, CHUNK), dst.reshape(NW, NCH, CHUNK)


def kernel(x1, edge_index1, x2, edge_index2, x3, edge_index3,
           W1, att_src1, att_dst1, b1,
           W2, att_src2, att_dst2, b2,
           W3, att_src3, att_dst3, b3,
           fnn_W, fnn_b):
    pad = jnp.zeros((NPAD - N, D), jnp.float32)
    tabs = []
    for x, ei, W, a_s, a_d in ((x1, edge_index1, W1, att_src1, att_dst1),
                               (x2, edge_index2, W2, att_src2, att_dst2),
                               (x3, edge_index3, W3, att_src3, att_dst3)):
        ms = jnp.pad(_att_fold(a_s), ((0, 0), (0, RW - HC - H)))
        md = jnp.pad(_att_fold(a_d), ((0, 0), (0, ADW - H)))
        wfull = jnp.concatenate([W, W @ ms], axis=1) @ _PERM
        wad = W @ md
        xpad = jnp.concatenate([x, pad], axis=0)
        t, ad = _build_tables(xpad, wfull, wad)
        t32 = lax.bitcast_convert_type(t.reshape(NPAD, TW, 2), jnp.int32)
        s, d = _edge_arrays(ei)
        tabs.extend([t32, ad, s, d])

    zsrc = jnp.zeros((ZR, RW), jnp.float32)
    p = _run_edges(tabs, zsrc)

    biases = jnp.stack([b1, b2, b3])
    exp9 = jnp.repeat(jnp.eye(H, dtype=jnp.float32), C, axis=1)
    return _finalize(p, biases, exp9, fnn_W, fnn_b.reshape(1, 2))


# speedup vs baseline: 1.1759x; 1.0012x over previous
import numpy as np

import jax
import jax.numpy as jnp
from jax import lax
from jax.experimental import pallas as pl
from jax.experimental.pallas import tpu as pltpu
from jax.experimental.pallas import tpu_sc as plsc

N = 10000
NPAD = 10016
D = 128
H = 9
C = 8
HC = 72
RW = 96
ADW = 16
E = 320000
EPRIME = E + N
NCORES = 2
NSUB = 16
NW = NCORES * NSUB
CHUNK = 128
ET = -(-(-(-EPRIME // NW)) // CHUNK) * CHUNK
EP = ET * NW
NCH = ET // CHUNK
ZR = NPAD // NSUB
TW = 48
NB = 4
TBLK = NPAD // NB


def _tables_body(x_ref, wf_ref, wad_ref, t_ref, ad_ref):
    x = x_ref[...]
    y = jnp.dot(x, wf_ref[...], preferred_element_type=jnp.float32)
    t_ref[...] = y.astype(jnp.bfloat16)
    ad_ref[...] = jnp.dot(x, wad_ref[...], preferred_element_type=jnp.float32)


def _build_tables(xpad, wfull, wad):
    return pl.pallas_call(
        _tables_body,
        grid=(NB,),
        in_specs=[
            pl.BlockSpec((TBLK, D), lambda i: (i, 0)),
            pl.BlockSpec((D, RW), lambda i: (0, 0)),
            pl.BlockSpec((D, ADW), lambda i: (0, 0)),
        ],
        out_specs=[
            pl.BlockSpec((TBLK, RW), lambda i: (i, 0)),
            pl.BlockSpec((TBLK, ADW), lambda i: (i, 0)),
        ],
        out_shape=[
            jax.ShapeDtypeStruct((NPAD, RW), jnp.bfloat16),
            jax.ShapeDtypeStruct((NPAD, ADW), jnp.float32),
        ],
    )(xpad, wfull, wad)


_GD = lax.GatherDimensionNumbers(
    offset_dims=(), collapsed_slice_dims=(0,), start_index_map=(0,))


def _vgather(w, idx):
    return lax.gather(w, idx[:, None], _GD, slice_sizes=(1,),
                      mode=lax.GatherScatterMode.PROMISE_IN_BOUNDS)
def _edge_kernel(t1, ad1, s1, d1, t2, ad2, s2, d2, t3, ad3, s3, d3, zsrc,
                 p_out, sidx2, didx2, rows_t, rows_ad, val, acc,
                 semg0, semg1, sems0, sems1):
    cid = lax.axis_index("c")
    sid = lax.axis_index("s")
    wid = cid * NSUB + sid

    io = lax.iota(jnp.int32, 16)
    i0 = io // 8
    i1 = i0 + 2
    i2 = i0 + 4
    i3 = i0 + 6
    i4 = jnp.full((16,), 8, jnp.int32)
    lt8 = io < 8
    lt9 = io < 9
    semg = (semg0, semg1)
    sems = (sems0, sems1)

    for g, (tg, adg, sg, dg) in enumerate(
            ((t1, ad1, s1, d1), (t2, ad2, s2, d2), (t3, ad3, s3, d3))):
        pltpu.sync_copy(zsrc, acc.at[pl.ds(sid * ZR, ZR)])
        pltpu.sync_copy(sg.at[wid], sidx2)
        pltpu.sync_copy(dg.at[wid], didx2)
        plsc.subcore_barrier()

        def issue(ch, b, tg=tg, adg=adg):
            pltpu.async_copy(tg.at[sidx2.at[ch]], rows_t.at[b], semg[b])
            pltpu.async_copy(adg.at[didx2.at[ch]], rows_ad.at[b], semg[b])

        def wait_gather(b, tg=tg, adg=adg):
            pltpu.make_async_copy(tg.at[pl.ds(0, CHUNK)],
                                  rows_t.at[b], semg[b]).wait()
            pltpu.make_async_copy(adg.at[pl.ds(0, CHUNK)],
                                  rows_ad.at[b], semg[b]).wait()

        def drain_scatter(b, tg=tg):
            pltpu.make_async_copy(tg.at[pl.ds(0, CHUNK)],
                                  val.at[b], sems[b]).wait()

        def compute(b):
            rt = rows_t.at[b]
            ra = rows_ad.at[b]
            vb = val.at[b]

            msk = jnp.int32(-65536)

            @plsc.parallel_loop(0, CHUNK, unroll=4)
            def edge_body(e):
                v01 = rt[e, pl.ds(0, 16)]
                v23 = rt[e, pl.ds(16, 16)]
                v45 = rt[e, pl.ds(32, 16)]
                g0 = plsc.bitcast(jnp.left_shift(v01, 16), jnp.float32)
                g1 = plsc.bitcast(jnp.bitwise_and(v01, msk), jnp.float32)
                g2 = plsc.bitcast(jnp.left_shift(v23, 16), jnp.float32)
                g3 = plsc.bitcast(jnp.bitwise_and(v23, msk), jnp.float32)
                g4 = plsc.bitcast(jnp.left_shift(v45, 16), jnp.float32)
                a_s = plsc.bitcast(jnp.bitwise_and(v45, msk), jnp.float32)
                a_d = ra[e, :]
                att = a_s + a_d
                att = jnp.maximum(att, 0.2 * att)
                w = jnp.exp(att)
                w0 = _vgather(w, i0)
                w1 = _vgather(w, i1)
                w2 = _vgather(w, i2)
                w3 = _vgather(w, i3)
                w4 = _vgather(w, i4)
                vb[e, pl.ds(0, 16)] = g0 * w0
                vb[e, pl.ds(16, 16)] = g1 * w1
                vb[e, pl.ds(32, 16)] = g2 * w2
                vb[e, pl.ds(48, 16)] = g3 * w3
                vb[e, pl.ds(64, 16)] = g4 * w4
                vb[e, pl.ds(80, 16)] = jnp.where(lt9, w, 0.0)

        def scatter(ch, b):
            pltpu.async_copy(val.at[b], acc.at[didx2.at[ch]], sems[b],
                             add=True)

        issue(0, 0)

        def pair_body(i, carry):
            ch0 = 2 * i
            issue(ch0 + 1, 1)
            wait_gather(0)

            @pl.when(i >= 1)
            def _():
                drain_scatter(0)
            compute(0)
            scatter(ch0, 0)

            @pl.when(ch0 + 2 < NCH)
            def _():
                issue(ch0 + 2, 0)
            wait_gather(1)

            @pl.when(i >= 1)
            def _():
                drain_scatter(1)
            compute(1)
            scatter(ch0 + 1, 1)
            return carry

        lax.fori_loop(0, NCH // 2, pair_body, 0)
        if NCH % 2:
            wait_gather(0)
            drain_scatter(0)
            compute(0)
            scatter(NCH - 1, 0)
        drain_scatter(0)
        drain_scatter(1)
        plsc.subcore_barrier()
        pltpu.sync_copy(acc.at[pl.ds(sid * ZR, ZR)],
                        p_out.at[g, cid, pl.ds(sid * ZR, ZR)])


def _run_edges(tabs, zsrc):
    mesh = plsc.VectorSubcoreMesh(core_axis_name="c", subcore_axis_name="s",
                                  num_cores=NCORES, num_subcores=NSUB)
    k = pl.kernel(
        _edge_kernel,
        mesh=mesh,
        compiler_params=pltpu.CompilerParams(use_tc_tiling_on_sc=False,
                                             needs_layout_passes=False),
        out_type=jax.ShapeDtypeStruct((3, NCORES, NPAD, RW), jnp.float32),
        scratch_types=[
            pltpu.VMEM((NCH, CHUNK), jnp.int32),
            pltpu.VMEM((NCH, CHUNK), jnp.int32),
            pltpu.VMEM((2, CHUNK, TW), jnp.int32),
            pltpu.VMEM((2, CHUNK, ADW), jnp.float32),
            pltpu.VMEM((2, CHUNK, RW), jnp.float32),
            pltpu.VMEM_SHARED((NPAD, RW), jnp.float32),
            pltpu.SemaphoreType.DMA,
            pltpu.SemaphoreType.DMA,
            pltpu.SemaphoreType.DMA,
            pltpu.SemaphoreType.DMA,
        ],
    )
    return k(*tabs, zsrc)


def _final_body(p_ref, bias_ref, exp9_ref, fw_ref, fb_ref, o_ref):
    p = p_ref[...]
    outs = []
    for g in range(3):
        num = p[g, 0, :, 0:HC] + p[g, 1, :, 0:HC]
        den = p[g, 0, :, 80:89] + p[g, 1, :, 80:89]
        rec = 1.0 / (den + 1e-16)
        rec_exp = jnp.dot(rec, exp9_ref[...],
                          preferred_element_type=jnp.float32)
        outs.append(num * rec_exp + bias_ref[g])
    xcat = jnp.concatenate(outs, axis=1)
    xcat = jnp.maximum(xcat, 0.0)
    logits = jnp.dot(xcat, fw_ref[...],
                     preferred_element_type=jnp.float32) + fb_ref[...]
    m = jnp.max(logits, axis=1, keepdims=True)
    ex = jnp.exp(logits - m)
    o_ref[...] = ex / jnp.sum(ex, axis=1, keepdims=True)


def _finalize(p, biases, exp9, fnn_w, fnn_b):
    blk = 2000
    return pl.pallas_call(
        _final_body,
        grid=(5,),
        in_specs=[
            pl.BlockSpec((3, NCORES, blk, RW), lambda i: (0, 0, i, 0)),
            pl.BlockSpec((3, HC), lambda i: (0, 0)),
            pl.BlockSpec((H, HC), lambda i: (0, 0)),
            pl.BlockSpec((3 * HC, 2), lambda i: (0, 0)),
            pl.BlockSpec((1, 2), lambda i: (0, 0)),
        ],
        out_specs=pl.BlockSpec((blk, 2), lambda i: (i, 0)),
        out_shape=jax.ShapeDtypeStruct((N, 2), jnp.float32),
    )(p, biases, exp9, fnn_w, fnn_b)


def _perm_matrix():
    p = np.zeros((RW, RW), np.float32)
    for k in range(16):
        p[k, 2 * k] = 1.0
        p[16 + k, 2 * k + 1] = 1.0
        p[32 + k, 32 + 2 * k] = 1.0
        p[48 + k, 32 + 2 * k + 1] = 1.0
    for k in range(8):
        p[64 + k, 64 + 2 * k] = 1.0
    for k in range(H):
        p[72 + k, 64 + 2 * k + 1] = 1.0
    return p


_PERM = _perm_matrix()


def _att_fold(att):
    a = att.reshape(H, C)
    return (a[:, :, None] * jnp.eye(H, dtype=jnp.float32)[:, None, :]
            ).reshape(HC, H)


def _edge_arrays(ei):
    loop = jnp.arange(N, dtype=jnp.int32)
    src = jnp.concatenate([ei[0].astype(jnp.int32), loop,
                           jnp.zeros((EP - EPRIME,), jnp.int32)])
    dst = jnp.concatenate([ei[1].astype(jnp.int32), loop,
                           jnp.full((EP - EPRIME,), N, jnp.int32)])
    return src.reshape(NW, NCH, CHUNK), dst.reshape(NW, NCH, CHUNK)


def kernel(x1, edge_index1, x2, edge_index2, x3, edge_index3,
           W1, att_src1, att_dst1, b1,
           W2, att_src2, att_dst2, b2,
           W3, att_src3, att_dst3, b3,
           fnn_W, fnn_b):
    pad = jnp.zeros((NPAD - N, D), jnp.float32)
    tabs = []
    for x, ei, W, a_s, a_d in ((x1, edge_index1, W1, att_src1, att_dst1),
                               (x2, edge_index2, W2, att_src2, att_dst2),
                               (x3, edge_index3, W3, att_src3, att_dst3)):
        ms = jnp.pad(_att_fold(a_s), ((0, 0), (0, RW - HC - H)))
        md = jnp.pad(_att_fold(a_d), ((0, 0), (0, ADW - H)))
        wfull = jnp.concatenate([W, W @ ms], axis=1) @ _PERM
        wad = W @ md
        xpad = jnp.concatenate([x, pad], axis=0)
        t, ad = _build_tables(xpad, wfull, wad)
        t32 = lax.bitcast_convert_type(t.reshape(NPAD, TW, 2), jnp.int32)
        s, d = _edge_arrays(ei)
        tabs.extend([t32, ad, s, d])

    zsrc = jnp.zeros((ZR, RW), jnp.float32)
    p = _run_edges(tabs, zsrc)

    biases = jnp.stack([b1, b2, b3])
    exp9 = jnp.repeat(jnp.eye(H, dtype=jnp.float32), C, axis=1)
    return _finalize(p, biases, exp9, fnn_W, fnn_b.reshape(1, 2))
